# baseline (speedup 1.0000x reference)
import functools

import jax
import jax.numpy as jnp
from jax import lax
from jax.experimental import pallas as pl
from jax.experimental.pallas import tpu as pltpu
from jax.experimental.pallas import tpu_sc as plsc

DIM = 64
BATCH = 4096
NUM_ROWS = 100000
BLK = 20480
NC = 2
NS = 16
L = 16
NW = NC * NS
BPW = BATCH // NW


def _tc_matvec(table_ref, w_ref, u_ref, b_ref, s_ref):
    c = jnp.sum(u_ref[:] * w_ref[pl.ds(0, DIM)]) + b_ref[0]
    w2 = w_ref[pl.ds(DIM, DIM)]
    t3 = table_ref[:].reshape(BLK // 128, 128, DIM)
    s2d = lax.dot_general(w2.reshape(1, DIM), t3,
                          (((1,), (2,)), ((), ())),
                          preferred_element_type=jnp.float32,
                          precision=lax.Precision.DEFAULT)
    s_ref[...] = (s2d + c).reshape(BLK // 128, 128)


def _sc_lookup(idx_hbm, s2_hbm, out_hbm, idx_v, hi_v, rows_v, out_v, sem):
    wid = lax.axis_index("s") * NC + lax.axis_index("c")
    base = wid * BPW

    pltpu.sync_copy(idx_hbm.at[pl.ds(base, BPW)], idx_v)
    for k in range(BPW // L):
        hi_v[pl.ds(k * L, L)] = lax.shift_right_logical(idx_v[pl.ds(k * L, L)], 7)
    pltpu.async_copy(s2_hbm.at[hi_v], rows_v, sem).wait()

    lane = lax.iota(jnp.int32, L)
    for g in range(BPW // L):
        col = idx_v[pl.ds(g * L, L)] & 127
        out_v[pl.ds(g * L, L)] = plsc.load_gather(rows_v, [g * L + lane, col])

    pltpu.sync_copy(out_v, out_hbm.at[pl.ds(base, BPW)])


@jax.jit
def kernel(item_idx, user_emb, item_emb, fc_w, fc_b):
    s = pl.pallas_call(
        _tc_matvec,
        grid=(5,),
        in_specs=[
            pl.BlockSpec((BLK, DIM), lambda i: (i, 0)),
            pl.BlockSpec((2 * DIM,), lambda i: (0,)),
            pl.BlockSpec((DIM,), lambda i: (0,)),
            pl.BlockSpec((1,), lambda i: (0,)),
        ],
        out_specs=pl.BlockSpec((BLK // 128, 128), lambda i: (i, 0)),
        out_shape=jax.ShapeDtypeStruct((5 * BLK // 128, 128), jnp.float32),
    )(item_emb, fc_w.reshape(2 * DIM), user_emb.reshape(DIM), fc_b)

    mesh = plsc.VectorSubcoreMesh(core_axis_name="c", subcore_axis_name="s")
    lookup = functools.partial(
        pl.kernel,
        mesh=mesh,
        compiler_params=pltpu.CompilerParams(needs_layout_passes=False,
                                             use_tc_tiling_on_sc=False),
        out_type=jax.ShapeDtypeStruct((BATCH,), jnp.float32),
        scratch_types=[
            pltpu.VMEM((BPW,), jnp.int32),
            pltpu.VMEM((BPW,), jnp.int32),
            pltpu.VMEM((BPW, 128), jnp.float32),
            pltpu.VMEM((BPW,), jnp.float32),
            pltpu.SemaphoreType.DMA,
        ],
    )(_sc_lookup)
    out = lookup(item_idx.astype(jnp.int32), s)
    return out.reshape(BATCH, 1)


# speedup vs baseline: 2.5138x; 2.5138x over previous
import functools

import jax
import jax.numpy as jnp
from jax import lax
from jax.experimental import pallas as pl
from jax.experimental.pallas import tpu as pltpu
from jax.experimental.pallas import tpu_sc as plsc

DIM = 64
BATCH = 4096
NUM_ROWS = 100000
BLKC = 25600
NC = 2
NS = 16
L = 16
NW = NC * NS
BPW = BATCH // NW


def _tc_matvec(tt_ref, w_ref, u_ref, b_ref, s_ref):
    c = jnp.sum(u_ref[:] * w_ref[pl.ds(0, DIM)]) + b_ref[0]
    w2 = w_ref[pl.ds(DIM, DIM)]
    s2d = lax.dot_general(w2.reshape(1, DIM), tt_ref[:],
                          (((1,), (0,)), ((), ())),
                          preferred_element_type=jnp.float32,
                          precision=lax.Precision.DEFAULT)
    s_ref[...] = (s2d + c).reshape(BLKC // 128, 128)


def _sc_lookup(idx_hbm, s2_hbm, out_hbm, idx_v, hi_v, rows_v, out_v, sem):
    wid = lax.axis_index("s") * NC + lax.axis_index("c")
    base = wid * BPW

    pltpu.sync_copy(idx_hbm.at[pl.ds(base, BPW)], idx_v)
    for k in range(BPW // L):
        hi_v[pl.ds(k * L, L)] = lax.shift_right_logical(idx_v[pl.ds(k * L, L)], 7)
    pltpu.async_copy(s2_hbm.at[hi_v], rows_v, sem).wait()

    lane = lax.iota(jnp.int32, L)
    for g in range(BPW // L):
        col = idx_v[pl.ds(g * L, L)] & 127
        out_v[pl.ds(g * L, L)] = plsc.load_gather(rows_v, [g * L + lane, col])

    pltpu.sync_copy(out_v, out_hbm.at[pl.ds(base, BPW)])


@jax.jit
def kernel(item_idx, user_emb, item_emb, fc_w, fc_b):
    s = pl.pallas_call(
        _tc_matvec,
        grid=(4,),
        in_specs=[
            pl.BlockSpec((DIM, BLKC), lambda i: (0, i)),
            pl.BlockSpec((2 * DIM,), lambda i: (0,)),
            pl.BlockSpec((DIM,), lambda i: (0,)),
            pl.BlockSpec((1,), lambda i: (0,)),
        ],
        out_specs=pl.BlockSpec((BLKC // 128, 128), lambda i: (i, 0)),
        out_shape=jax.ShapeDtypeStruct((4 * BLKC // 128, 128), jnp.float32),
    )(item_emb.T, fc_w.reshape(2 * DIM), user_emb.reshape(DIM), fc_b)

    mesh = plsc.VectorSubcoreMesh(core_axis_name="c", subcore_axis_name="s")
    lookup = functools.partial(
        pl.kernel,
        mesh=mesh,
        compiler_params=pltpu.CompilerParams(needs_layout_passes=False,
                                             use_tc_tiling_on_sc=False),
        out_type=jax.ShapeDtypeStruct((BATCH,), jnp.float32),
        scratch_types=[
            pltpu.VMEM((BPW,), jnp.int32),
            pltpu.VMEM((BPW,), jnp.int32),
            pltpu.VMEM((BPW, 128), jnp.float32),
            pltpu.VMEM((BPW,), jnp.float32),
            pltpu.SemaphoreType.DMA,
        ],
    )(_sc_lookup)
    out = lookup(item_idx.astype(jnp.int32), s)
    return out.reshape(BATCH, 1)


# speedup vs baseline: 2.6115x; 1.0389x over previous
import functools

import jax
import jax.numpy as jnp
from jax import lax
from jax.experimental import pallas as pl
from jax.experimental.pallas import tpu as pltpu
from jax.experimental.pallas import tpu_sc as plsc

DIM = 64
BATCH = 4096
NUM_ROWS = 100000
BLKC = 25600
NUM_ROWS_PAD = 4 * BLKC
NC = 2
NS = 16
L = 16
NW = NC * NS
BPW = BATCH // NW


def _tc_matvec(tt_ref, w_ref, u_ref, b_ref, s_ref):
    c = jnp.sum(u_ref[:] * w_ref[pl.ds(0, DIM)]) + b_ref[0]
    w2 = w_ref[pl.ds(DIM, DIM)]
    s2d = lax.dot_general(w2.reshape(1, DIM), tt_ref[:],
                          (((1,), (0,)), ((), ())),
                          preferred_element_type=jnp.float32,
                          precision=lax.Precision.DEFAULT)
    s_ref[...] = (s2d + c).reshape(BLKC // 128, 128)


def _sc_lookup(idx_hbm, s2_hbm, out_hbm, idx_v, hi_v, rows_v, out_v, sem):
    wid = lax.axis_index("s") * NC + lax.axis_index("c")
    base = wid * BPW

    pltpu.sync_copy(idx_hbm.at[pl.ds(base, BPW)], idx_v)
    for k in range(BPW // L):
        hi_v[pl.ds(k * L, L)] = lax.shift_right_logical(idx_v[pl.ds(k * L, L)], 4)
    pltpu.async_copy(s2_hbm.at[hi_v], rows_v, sem).wait()

    lane = lax.iota(jnp.int32, L)
    for g in range(BPW // L):
        col = idx_v[pl.ds(g * L, L)] & 15
        out_v[pl.ds(g * L, L)] = plsc.load_gather(rows_v, [g * L + lane, col])

    pltpu.sync_copy(out_v, out_hbm.at[pl.ds(base, BPW)])


@jax.jit
def kernel(item_idx, user_emb, item_emb, fc_w, fc_b):
    s = pl.pallas_call(
        _tc_matvec,
        grid=(4,),
        in_specs=[
            pl.BlockSpec((DIM, BLKC), lambda i: (0, i)),
            pl.BlockSpec((2 * DIM,), lambda i: (0,)),
            pl.BlockSpec((DIM,), lambda i: (0,)),
            pl.BlockSpec((1,), lambda i: (0,)),
        ],
        out_specs=pl.BlockSpec((BLKC // 128, 128), lambda i: (i, 0)),
        out_shape=jax.ShapeDtypeStruct((4 * BLKC // 128, 128), jnp.float32),
    )(item_emb.T, fc_w.reshape(2 * DIM), user_emb.reshape(DIM), fc_b)

    mesh = plsc.VectorSubcoreMesh(core_axis_name="c", subcore_axis_name="s")
    lookup = functools.partial(
        pl.kernel,
        mesh=mesh,
        compiler_params=pltpu.CompilerParams(needs_layout_passes=False,
                                             use_tc_tiling_on_sc=False),
        out_type=jax.ShapeDtypeStruct((BATCH,), jnp.float32),
        scratch_types=[
            pltpu.VMEM((BPW,), jnp.int32),
            pltpu.VMEM((BPW,), jnp.int32),
            pltpu.VMEM((BPW, L), jnp.float32),
            pltpu.VMEM((BPW,), jnp.float32),
            pltpu.SemaphoreType.DMA,
        ],
    )(_sc_lookup)
    out = lookup(item_idx.astype(jnp.int32), s.reshape(NUM_ROWS_PAD // L, L))
    return out.reshape(BATCH, 1)
